# baseline (speedup 1.0000x reference)
import functools

import jax
import jax.numpy as jnp
from jax import lax
from jax.experimental import pallas as pl
from jax.experimental.pallas import tpu as pltpu
from jax.experimental.pallas import tpu_sc as plsc

N_USERS = 25000
N_ITEMS = 25000
D = 64
N_LAYERS = 3
E = 800000

NC = 2
NS = 16
L = 16

CLEN = 128
SUB = 16
K = SUB * CLEN
CH = 26
E_PAD = NS * CH * K
E_ALLOC = (NS * CH + 1) * K
NBUF = 2

R_TILE = 1600
HALF = NS * R_TILE
X_ROWS = NC * HALF
TRASH = N_USERS


def _layer_body(x_hbm, src_hbm, dl_hbm, vals_hbm, out_hbm,
                src_v, dli_v, vals_v, rows_v, acc_sh,
                isem0, isem1, gsems, ssems):
    s = lax.axis_index("c")
    t = lax.axis_index("s")
    isems = (isem0, isem1)

    def fire_idx(ci, p):
        pltpu.async_copy(src_hbm.at[ci], src_v.at[p], isems[p])
        pltpu.async_copy(dl_hbm.at[s, ci], dli_v.at[p], isems[p])
        pltpu.async_copy(vals_hbm.at[ci], vals_v.at[p], isems[p])

    def drain_idx(p):
        pltpu.make_async_copy(src_hbm.at[0], src_v.at[p], isems[p]).wait()
        pltpu.make_async_copy(dl_hbm.at[0, 0], dli_v.at[p], isems[p]).wait()
        pltpu.make_async_copy(vals_hbm.at[0], vals_v.at[p], isems[p]).wait()

    fire_idx(t * CH, 0)

    zbuf = rows_v.at[0]

    def zrow(i, c):
        for k in range(D // L):
            zbuf[i, pl.ds(k * L, L)] = jnp.zeros((L,), jnp.float32)
        return c
    lax.fori_loop(0, CLEN, zrow, 0)
    base_acc = t * R_TILE
    for i in range(R_TILE // CLEN):
        pltpu.sync_copy(zbuf, acc_sh.at[pl.ds(base_acc + i * CLEN, CLEN)])
    rem = R_TILE % CLEN
    if rem:
        pltpu.sync_copy(zbuf.at[pl.ds(0, rem)],
                        acc_sh.at[pl.ds(base_acc + (R_TILE // CLEN) * CLEN, rem)])
    plsc.subcore_barrier()

    def scale(rbuf, p, j):
        def scale16(g, cc):
            v16 = vals_v[p, pl.ds(j * CLEN + g * L, L)]
            for e in range(L):
                r = g * L + e
                v = v16[e]
                for k in range(D // L):
                    sl = pl.ds(k * L, L)
                    rbuf[r, sl] = rbuf[r, sl] * v
            return cc
        lax.fori_loop(0, CLEN // L, scale16, 0)

    def process_chunk(p):
        gd = [None] * NBUF
        sd = [None] * NBUF
        for j in range(min(NBUF - 1, SUB)):
            gd[j] = pltpu.async_copy(
                x_hbm.at[src_v.at[p, j]], rows_v.at[j], gsems.at[j])
        for j in range(SUB):
            a = j % NBUF
            pre = j + NBUF - 1
            if pre < SUB:
                b = pre % NBUF
                if sd[b] is not None:
                    sd[b].wait()
                    sd[b] = None
                gd[b] = pltpu.async_copy(
                    x_hbm.at[src_v.at[p, pre]], rows_v.at[b], gsems.at[b])
            gd[a].wait()
            scale(rows_v.at[a], p, j)
            sd[a] = pltpu.async_copy(
                rows_v.at[a], acc_sh.at[dli_v.at[p, j]], ssems.at[a], add=True)
        for a in range(NBUF):
            if sd[a] is not None:
                sd[a].wait()

    def pair(i, carry):
        c0 = t * CH + 2 * i
        drain_idx(0)
        fire_idx(c0 + 1, 1)
        process_chunk(0)
        drain_idx(1)
        fire_idx(c0 + 2, 0)
        process_chunk(1)
        return carry
    lax.fori_loop(0, CH // 2, pair, 0)
    drain_idx(0)

    plsc.subcore_barrier()
    pltpu.sync_copy(acc_sh.at[pl.ds(base_acc, R_TILE)],
                    out_hbm.at[pl.ds(s * HALF + base_acc, R_TILE)])


_layer = functools.partial(
    pl.kernel,
    out_type=jax.ShapeDtypeStruct((X_ROWS, D), jnp.float32),
    mesh=plsc.VectorSubcoreMesh(core_axis_name="c", subcore_axis_name="s",
                                num_cores=NC, num_subcores=NS),
    scratch_types=[
        pltpu.VMEM((2, SUB, CLEN), jnp.int32),
        pltpu.VMEM((2, SUB, CLEN), jnp.int32),
        pltpu.VMEM((2, K), jnp.float32),
        pltpu.VMEM((NBUF, CLEN, D), jnp.float32),
        pltpu.VMEM_SHARED((HALF, D), jnp.float32),
        pltpu.SemaphoreType.DMA,
        pltpu.SemaphoreType.DMA,
        pltpu.SemaphoreType.DMA((NBUF,)),
        pltpu.SemaphoreType.DMA((NBUF,)),
    ],
    compiler_params=pltpu.CompilerParams(use_tc_tiling_on_sc=False),
)(_layer_body)


def kernel(adj_indices, adj_values, user_emb, item_emb):
    dst = adj_indices[0].astype(jnp.int32)
    src = adj_indices[1].astype(jnp.int32)
    vals = adj_values.astype(jnp.float32)

    pad = E_ALLOC - E
    dst = jnp.concatenate([dst, jnp.zeros((pad,), jnp.int32)])
    src = jnp.concatenate([src, jnp.zeros((pad,), jnp.int32)])
    vals = jnp.concatenate([vals, jnp.zeros((pad,), jnp.float32)])

    srcp = src + jnp.where(src >= N_USERS, HALF - N_USERS, 0).astype(jnp.int32)
    dl0 = jnp.where(dst < N_USERS, dst, TRASH).astype(jnp.int32)
    dl1 = jnp.where(dst >= N_USERS, dst - N_USERS, TRASH).astype(jnp.int32)

    src3 = srcp.reshape(NS * CH + 1, SUB, CLEN)
    dl4 = jnp.stack([dl0, dl1]).reshape(NC, NS * CH + 1, SUB, CLEN)
    vals2 = vals.reshape(NS * CH + 1, K)

    zpad = jnp.zeros((HALF - N_USERS, D), jnp.float32)
    x = jnp.concatenate([user_emb, zpad, item_emb, zpad], axis=0)

    acc = x
    for _ in range(N_LAYERS):
        x = _layer(x, src3, dl4, vals2)
        acc = acc + x
    out = acc * (1.0 / (N_LAYERS + 1))
    return (out[:N_USERS], out[HALF:HALF + N_ITEMS])


# speedup vs baseline: 1.4052x; 1.4052x over previous
import functools

import jax
import jax.numpy as jnp
from jax import lax
from jax.experimental import pallas as pl
from jax.experimental.pallas import tpu as pltpu
from jax.experimental.pallas import tpu_sc as plsc

N_USERS = 25000
N_ITEMS = 25000
D = 64
N_LAYERS = 3
E = 800000

NC = 2
NS = 16
L = 16

CLEN = 128
SUB = 8
K = SUB * CLEN
CH = 50
E_ALLOC = (NS * CH + 1) * K

R_TILE = 1600
HALF = NS * R_TILE
X_ROWS = NC * HALF
TRASH = N_USERS


def _layer_body(x_hbm, src_hbm, dl_hbm, vals_hbm, out_hbm,
                src_v, dli_v, vals_v, rows_v, acc_sh,
                isem0, isem1, gsem0, gsem1):
    s = lax.axis_index("c")
    t = lax.axis_index("s")
    isems = (isem0, isem1)
    gsems = (gsem0, gsem1)

    def fire_idx(ci, p):
        pltpu.async_copy(src_hbm.at[ci], src_v.at[p], isems[p])
        pltpu.async_copy(dl_hbm.at[s, ci], dli_v.at[p], isems[p])
        pltpu.async_copy(vals_hbm.at[ci], vals_v.at[p], isems[p])

    def drain_idx(p):
        pltpu.make_async_copy(src_hbm.at[0], src_v.at[p], isems[p]).wait()
        pltpu.make_async_copy(dl_hbm.at[0, 0], dli_v.at[p], isems[p]).wait()
        pltpu.make_async_copy(vals_hbm.at[0], vals_v.at[p], isems[p]).wait()

    def fire_gather(p, j, b):
        pltpu.async_copy(x_hbm.at[src_v.at[p, j]], rows_v.at[b], gsems[b])

    def wait_gather(b):
        pltpu.make_async_copy(x_hbm.at[pl.ds(0, CLEN)], rows_v.at[b],
                              gsems[b]).wait()

    fire_idx(t * CH, 0)

    zbuf = rows_v.at[0]

    def zrow(i, c):
        for k in range(D // L):
            zbuf[i, pl.ds(k * L, L)] = jnp.zeros((L,), jnp.float32)
        return c
    lax.fori_loop(0, CLEN, zrow, 0)
    base_acc = t * R_TILE
    for i in range(R_TILE // CLEN):
        pltpu.sync_copy(zbuf, acc_sh.at[pl.ds(base_acc + i * CLEN, CLEN)])
    plsc.subcore_barrier()

    def scale(rbuf, p, j):
        def scale16(g, cc):
            v16 = vals_v[p, pl.ds(j * CLEN + g * L, L)]
            for e in range(L):
                r = g * L + e
                v = v16[e]
                for k in range(D // L):
                    sl = pl.ds(k * L, L)
                    rbuf[r, sl] = rbuf[r, sl] * v
            return cc
        lax.fori_loop(0, CLEN // L, scale16, 0)

    def do_sub(p, j, b):
        wait_gather(b)
        scale(rows_v.at[b], p, j)
        pltpu.sync_copy(rows_v.at[b], acc_sh.at[dli_v.at[p, j]], add=True)

    def process_chunk(p):
        fire_gather(p, 0, 0)

        def jpair(q, carry):
            j0 = 2 * q
            fire_gather(p, j0 + 1, 1)
            do_sub(p, j0, 0)
            fire_gather(p, jnp.minimum(j0 + 2, SUB - 1), 0)
            do_sub(p, j0 + 1, 1)
            return carry
        lax.fori_loop(0, SUB // 2, jpair, 0)
        wait_gather(0)

    def pair(i, carry):
        c0 = t * CH + 2 * i
        drain_idx(0)
        fire_idx(c0 + 1, 1)
        process_chunk(0)
        drain_idx(1)
        fire_idx(c0 + 2, 0)
        process_chunk(1)
        return carry
    lax.fori_loop(0, CH // 2, pair, 0)
    drain_idx(0)

    plsc.subcore_barrier()
    pltpu.sync_copy(acc_sh.at[pl.ds(base_acc, R_TILE)],
                    out_hbm.at[pl.ds(s * HALF + base_acc, R_TILE)])


_layer = functools.partial(
    pl.kernel,
    out_type=jax.ShapeDtypeStruct((X_ROWS, D), jnp.float32),
    mesh=plsc.VectorSubcoreMesh(core_axis_name="c", subcore_axis_name="s",
                                num_cores=NC, num_subcores=NS),
    scratch_types=[
        pltpu.VMEM((2, SUB, CLEN), jnp.int32),
        pltpu.VMEM((2, SUB, CLEN), jnp.int32),
        pltpu.VMEM((2, K), jnp.float32),
        pltpu.VMEM((2, CLEN, D), jnp.float32),
        pltpu.VMEM_SHARED((HALF, D), jnp.float32),
        pltpu.SemaphoreType.DMA,
        pltpu.SemaphoreType.DMA,
        pltpu.SemaphoreType.DMA,
        pltpu.SemaphoreType.DMA,
    ],
    compiler_params=pltpu.CompilerParams(use_tc_tiling_on_sc=False),
)(_layer_body)


def kernel(adj_indices, adj_values, user_emb, item_emb):
    dst = adj_indices[0].astype(jnp.int32)
    src = adj_indices[1].astype(jnp.int32)
    vals = adj_values.astype(jnp.float32)

    pad = E_ALLOC - E
    dst = jnp.concatenate([dst, jnp.zeros((pad,), jnp.int32)])
    src = jnp.concatenate([src, jnp.zeros((pad,), jnp.int32)])
    vals = jnp.concatenate([vals, jnp.zeros((pad,), jnp.float32)])

    srcp = src + jnp.where(src >= N_USERS, HALF - N_USERS, 0).astype(jnp.int32)
    dl0 = jnp.where(dst < N_USERS, dst, TRASH).astype(jnp.int32)
    dl1 = jnp.where(dst >= N_USERS, dst - N_USERS, TRASH).astype(jnp.int32)

    src3 = srcp.reshape(NS * CH + 1, SUB, CLEN)
    dl4 = jnp.stack([dl0, dl1]).reshape(NC, NS * CH + 1, SUB, CLEN)
    vals2 = vals.reshape(NS * CH + 1, K)

    zpad = jnp.zeros((HALF - N_USERS, D), jnp.float32)
    x = jnp.concatenate([user_emb, zpad, item_emb, zpad], axis=0)

    acc = x
    for _ in range(N_LAYERS):
        x = _layer(x, src3, dl4, vals2)
        acc = acc + x
    out = acc * (1.0 / (N_LAYERS + 1))
    return (out[:N_USERS], out[HALF:HALF + N_ITEMS])


# speedup vs baseline: 1.5612x; 1.1110x over previous
import functools

import jax
import jax.numpy as jnp
from jax import lax
from jax.experimental import pallas as pl
from jax.experimental.pallas import tpu as pltpu
from jax.experimental.pallas import tpu_sc as plsc

N_USERS = 25000
N_ITEMS = 25000
D = 64
N_LAYERS = 3
E = 800000

NC = 2
NS = 16
L = 16

CLEN = 128
SUB = 6
K = SUB * CLEN
CH = 67
E_ALLOC = (NS * CH + 2) * K

R_TILE = 1568
HALF = NS * R_TILE
X_ROWS = NC * HALF
TRASH = N_USERS


def _layer_body(x_hbm, src_hbm, dl_hbm, vals_hbm, out_hbm,
                src_v, dli_v, vals_v, rows_v, acc_sh,
                isem0, isem1, gsems, ssems):
    s = lax.axis_index("c")
    t = lax.axis_index("s")
    isems = (isem0, isem1)

    def fire_idx(ci, p):
        pltpu.async_copy(src_hbm.at[ci], src_v.at[p], isems[p])
        pltpu.async_copy(dl_hbm.at[s, ci], dli_v.at[p], isems[p])
        pltpu.async_copy(vals_hbm.at[ci], vals_v.at[p], isems[p])

    def drain_idx(p):
        pltpu.make_async_copy(src_hbm.at[0], src_v.at[p], isems[p]).wait()
        pltpu.make_async_copy(dl_hbm.at[0, 0], dli_v.at[p], isems[p]).wait()
        pltpu.make_async_copy(vals_hbm.at[0], vals_v.at[p], isems[p]).wait()

    def fire_g(p, j, b):
        return pltpu.async_copy(x_hbm.at[src_v.at[p, j]], rows_v.at[b],
                                gsems.at[b])

    def fire_s(p, j, b):
        return pltpu.async_copy(rows_v.at[b], acc_sh.at[dli_v.at[p, j]],
                                ssems.at[b], add=True)

    c_base = t * CH
    fire_idx(c_base, 0)

    zbuf = rows_v.at[0]

    def zrow(i, c):
        for k in range(D // L):
            zbuf[i, pl.ds(k * L, L)] = jnp.zeros((L,), jnp.float32)
        return c
    lax.fori_loop(0, CLEN, zrow, 0)
    base_acc = t * R_TILE
    for i in range(R_TILE // CLEN):
        pltpu.sync_copy(zbuf, acc_sh.at[pl.ds(base_acc + i * CLEN, CLEN)])
    rem = R_TILE % CLEN
    if rem:
        pltpu.sync_copy(zbuf.at[pl.ds(0, rem)],
                        acc_sh.at[pl.ds(base_acc + (R_TILE // CLEN) * CLEN, rem)])
    plsc.subcore_barrier()

    def scale(pv, j, b):
        rbuf = rows_v.at[b]

        def scale16(g, cc):
            v16 = vals_v[pv, pl.ds(j * CLEN + g * L, L)]
            for e in range(L):
                r = g * L + e
                v = v16[e]
                for k in range(D // L):
                    sl = pl.ds(k * L, L)
                    rbuf[r, sl] = rbuf[r, sl] * v
            return cc
        lax.fori_loop(0, CLEN // L, scale16, 0)

    def ring_pass(subs, hooks):
        n = len(subs)
        gd = [None] * 3
        sd = [None] * 3
        for k in range(min(2, n)):
            p, j = subs[k]
            gd[k] = fire_g(p, j, k)
        for k, (p, j) in enumerate(subs):
            b = k % 3
            gd[b].wait()
            scale(p, j, b)
            if k + 2 < n:
                nb = (k + 2) % 3
                if sd[nb] is not None:
                    sd[nb].wait()
                    sd[nb] = None
                if k in hooks:
                    hooks[k]()
                pn, jn = subs[k + 2]
                gd[nb] = fire_g(pn, jn, nb)
            sd[b] = fire_s(p, j, b)
        for b in range(3):
            if sd[b] is not None:
                sd[b].wait()

    drain_idx(0)
    fire_idx(c_base + 1, 1)
    ring_pass([(0, j) for j in range(SUB)], {})
    fire_idx(c_base + 2, 0)

    def pair(i, carry):
        ca = c_base + 2 * i + 1
        drain_idx(1)
        subs = [(1, j) for j in range(SUB)] + [(0, j) for j in range(SUB)]
        hooks = {
            3: lambda: drain_idx(0),
            7: lambda: fire_idx(ca + 2, 1),
        }
        ring_pass(subs, hooks)
        fire_idx(ca + 3, 0)
        return carry
    lax.fori_loop(0, (CH - 1) // 2, pair, 0)

    drain_idx(1)
    drain_idx(0)

    plsc.subcore_barrier()
    pltpu.sync_copy(acc_sh.at[pl.ds(base_acc, R_TILE)],
                    out_hbm.at[pl.ds(s * HALF + base_acc, R_TILE)])


_layer = functools.partial(
    pl.kernel,
    out_type=jax.ShapeDtypeStruct((X_ROWS, D), jnp.float32),
    mesh=plsc.VectorSubcoreMesh(core_axis_name="c", subcore_axis_name="s",
                                num_cores=NC, num_subcores=NS),
    scratch_types=[
        pltpu.VMEM((2, SUB, CLEN), jnp.int32),
        pltpu.VMEM((2, SUB, CLEN), jnp.int32),
        pltpu.VMEM((2, K), jnp.float32),
        pltpu.VMEM((3, CLEN, D), jnp.float32),
        pltpu.VMEM_SHARED((HALF, D), jnp.float32),
        pltpu.SemaphoreType.DMA,
        pltpu.SemaphoreType.DMA,
        pltpu.SemaphoreType.DMA((3,)),
        pltpu.SemaphoreType.DMA((3,)),
    ],
    compiler_params=pltpu.CompilerParams(use_tc_tiling_on_sc=False),
)(_layer_body)


def kernel(adj_indices, adj_values, user_emb, item_emb):
    dst = adj_indices[0].astype(jnp.int32)
    src = adj_indices[1].astype(jnp.int32)
    vals = adj_values.astype(jnp.float32)

    pad = E_ALLOC - E
    dst = jnp.concatenate([dst, jnp.zeros((pad,), jnp.int32)])
    src = jnp.concatenate([src, jnp.zeros((pad,), jnp.int32)])
    vals = jnp.concatenate([vals, jnp.zeros((pad,), jnp.float32)])

    srcp = src + jnp.where(src >= N_USERS, HALF - N_USERS, 0).astype(jnp.int32)
    dl0 = jnp.where(dst < N_USERS, dst, TRASH).astype(jnp.int32)
    dl1 = jnp.where(dst >= N_USERS, dst - N_USERS, TRASH).astype(jnp.int32)

    src3 = srcp.reshape(NS * CH + 2, SUB, CLEN)
    dl4 = jnp.stack([dl0, dl1]).reshape(NC, NS * CH + 2, SUB, CLEN)
    vals2 = vals.reshape(NS * CH + 2, K)

    zpad = jnp.zeros((HALF - N_USERS, D), jnp.float32)
    x = jnp.concatenate([user_emb, zpad, item_emb, zpad], axis=0)

    acc = x
    for _ in range(N_LAYERS):
        x = _layer(x, src3, dl4, vals2)
        acc = acc + x
    out = acc * (1.0 / (N_LAYERS + 1))
    return (out[:N_USERS], out[HALF:HALF + N_ITEMS])


# speedup vs baseline: 1.6916x; 1.0835x over previous
import functools

import jax
import jax.numpy as jnp
from jax import lax
from jax.experimental import pallas as pl
from jax.experimental.pallas import tpu as pltpu
from jax.experimental.pallas import tpu_sc as plsc

N_USERS = 25000
N_ITEMS = 25000
D = 64
N_LAYERS = 3
E = 800000

NC = 2
NS = 16
L = 16

CLEN = 128
SUB = 6
K = SUB * CLEN
CH = 67
E_ALLOC = (NS * CH + 2) * K

R_TILE = 1568
HALF = NS * R_TILE
X_ROWS = NC * HALF
TRASH = N_USERS


def _layer_body(x_hbm, src_hbm, dl_hbm, vals_hbm, out_hbm,
                src_v, dli_v, vals_v, rows_v, acc_sh,
                isem0, isem1, gsems, ssems):
    s = lax.axis_index("c")
    t = lax.axis_index("s")
    isems = (isem0, isem1)

    def fire_idx(ci, p):
        pltpu.async_copy(src_hbm.at[ci], src_v.at[p], isems[p])
        pltpu.async_copy(dl_hbm.at[s, ci], dli_v.at[p], isems[p])
        pltpu.async_copy(vals_hbm.at[ci], vals_v.at[p], isems[p])

    def drain_idx(p):
        pltpu.make_async_copy(src_hbm.at[0], src_v.at[p], isems[p]).wait()
        pltpu.make_async_copy(dl_hbm.at[0, 0], dli_v.at[p], isems[p]).wait()
        pltpu.make_async_copy(vals_hbm.at[0], vals_v.at[p], isems[p]).wait()

    def fire_g(p, j, b):
        return pltpu.async_copy(x_hbm.at[src_v.at[p, j]], rows_v.at[b],
                                gsems.at[b])

    def fire_s(p, j, b):
        return pltpu.async_copy(rows_v.at[b], acc_sh.at[dli_v.at[p, j]],
                                ssems.at[b], add=True)

    c_base = t * CH
    fire_idx(c_base, 0)

    zbuf = rows_v.at[0]

    def zrow(i, c):
        for k in range(D // L):
            zbuf[i, pl.ds(k * L, L)] = jnp.zeros((L,), jnp.float32)
        return c
    lax.fori_loop(0, CLEN, zrow, 0)
    base_acc = t * R_TILE
    for i in range(R_TILE // CLEN):
        pltpu.sync_copy(zbuf, acc_sh.at[pl.ds(base_acc + i * CLEN, CLEN)])
    rem = R_TILE % CLEN
    if rem:
        pltpu.sync_copy(zbuf.at[pl.ds(0, rem)],
                        acc_sh.at[pl.ds(base_acc + (R_TILE // CLEN) * CLEN, rem)])
    plsc.subcore_barrier()

    def scale(pv, j, b):
        rbuf = rows_v.at[b]

        def scale16(g, cc):
            v16 = vals_v[pv, pl.ds(j * CLEN + g * L, L)]
            for half in range(2):
                prods = []
                for e8 in range(8):
                    e = half * 8 + e8
                    r = g * L + e
                    v = v16[e]
                    for k in range(D // L):
                        prods.append((r, k, rbuf[r, pl.ds(k * L, L)] * v))
                for r, k, pr in prods:
                    rbuf[r, pl.ds(k * L, L)] = pr
            return cc
        lax.fori_loop(0, CLEN // L, scale16, 0)

    def ring_pass(subs, hooks):
        n = len(subs)
        gd = [None] * 3
        sd = [None] * 3
        for k in range(min(2, n)):
            p, j = subs[k]
            gd[k] = fire_g(p, j, k)
        for k, (p, j) in enumerate(subs):
            b = k % 3
            gd[b].wait()
            scale(p, j, b)
            if k + 2 < n:
                nb = (k + 2) % 3
                if sd[nb] is not None:
                    sd[nb].wait()
                    sd[nb] = None
                if k in hooks:
                    hooks[k]()
                pn, jn = subs[k + 2]
                gd[nb] = fire_g(pn, jn, nb)
            sd[b] = fire_s(p, j, b)
        for b in range(3):
            if sd[b] is not None:
                sd[b].wait()

    drain_idx(0)
    fire_idx(c_base + 1, 1)
    ring_pass([(0, j) for j in range(SUB)], {})
    fire_idx(c_base + 2, 0)

    def pair(i, carry):
        ca = c_base + 2 * i + 1
        drain_idx(1)
        subs = [(1, j) for j in range(SUB)] + [(0, j) for j in range(SUB)]
        hooks = {
            3: lambda: drain_idx(0),
            7: lambda: fire_idx(ca + 2, 1),
        }
        ring_pass(subs, hooks)
        fire_idx(ca + 3, 0)
        return carry
    lax.fori_loop(0, (CH - 1) // 2, pair, 0)

    drain_idx(1)
    drain_idx(0)

    plsc.subcore_barrier()
    pltpu.sync_copy(acc_sh.at[pl.ds(base_acc, R_TILE)],
                    out_hbm.at[pl.ds(s * HALF + base_acc, R_TILE)])


_layer = functools.partial(
    pl.kernel,
    out_type=jax.ShapeDtypeStruct((X_ROWS, D), jnp.float32),
    mesh=plsc.VectorSubcoreMesh(core_axis_name="c", subcore_axis_name="s",
                                num_cores=NC, num_subcores=NS),
    scratch_types=[
        pltpu.VMEM((2, SUB, CLEN), jnp.int32),
        pltpu.VMEM((2, SUB, CLEN), jnp.int32),
        pltpu.VMEM((2, K), jnp.float32),
        pltpu.VMEM((3, CLEN, D), jnp.float32),
        pltpu.VMEM_SHARED((HALF, D), jnp.float32),
        pltpu.SemaphoreType.DMA,
        pltpu.SemaphoreType.DMA,
        pltpu.SemaphoreType.DMA((3,)),
        pltpu.SemaphoreType.DMA((3,)),
    ],
    compiler_params=pltpu.CompilerParams(use_tc_tiling_on_sc=False),
)(_layer_body)


def kernel(adj_indices, adj_values, user_emb, item_emb):
    dst = adj_indices[0].astype(jnp.int32)
    src = adj_indices[1].astype(jnp.int32)
    vals = adj_values.astype(jnp.float32)

    pad = E_ALLOC - E
    dst = jnp.concatenate([dst, jnp.zeros((pad,), jnp.int32)])
    src = jnp.concatenate([src, jnp.zeros((pad,), jnp.int32)])
    vals = jnp.concatenate([vals, jnp.zeros((pad,), jnp.float32)])

    srcp = src + jnp.where(src >= N_USERS, HALF - N_USERS, 0).astype(jnp.int32)
    dl0 = jnp.where(dst < N_USERS, dst, TRASH).astype(jnp.int32)
    dl1 = jnp.where(dst >= N_USERS, dst - N_USERS, TRASH).astype(jnp.int32)

    src3 = srcp.reshape(NS * CH + 2, SUB, CLEN)
    dl4 = jnp.stack([dl0, dl1]).reshape(NC, NS * CH + 2, SUB, CLEN)
    vals2 = vals.reshape(NS * CH + 2, K)

    zpad = jnp.zeros((HALF - N_USERS, D), jnp.float32)
    x = jnp.concatenate([user_emb, zpad, item_emb, zpad], axis=0)

    acc = x
    for _ in range(N_LAYERS):
        x = _layer(x, src3, dl4, vals2)
        acc = acc + x
    out = acc * (1.0 / (N_LAYERS + 1))
    return (out[:N_USERS], out[HALF:HALF + N_ITEMS])


# speedup vs baseline: 3.2924x; 1.9464x over previous
import functools

import jax
import jax.numpy as jnp
from jax import lax
from jax.experimental import pallas as pl
from jax.experimental.pallas import tpu as pltpu
from jax.experimental.pallas import tpu_sc as plsc

N_USERS = 25000
N_ITEMS = 25000
D = 64
N_LAYERS = 3
E = 800000

NC = 2
NS = 16
L = 16

CLEN = 128
SUB = 8
K = SUB * CLEN
CH = 49
E_ALLOC = (NS * CH + 2) * K

R_TILE = 1568
HALF = NS * R_TILE
X_ROWS = NC * HALF
TRASH = N_USERS


def _layer_body(x_hbm, src_hbm, dl_hbm, vals_hbm, out_hbm,
                src_v, dli_v, vals_v, rows_v, acc_sh,
                isem0, isem1, gsems, ssems):
    s = lax.axis_index("c")
    t = lax.axis_index("s")
    isems = (isem0, isem1)

    def fire_idx(ci, p):
        pltpu.async_copy(src_hbm.at[ci], src_v.at[p], isems[p])
        pltpu.async_copy(dl_hbm.at[s, ci], dli_v.at[p], isems[p])
        pltpu.async_copy(vals_hbm.at[ci], vals_v.at[p], isems[p])

    def drain_idx(p):
        pltpu.make_async_copy(src_hbm.at[0], src_v.at[p], isems[p]).wait()
        pltpu.make_async_copy(dl_hbm.at[0, 0], dli_v.at[p], isems[p]).wait()
        pltpu.make_async_copy(vals_hbm.at[0], vals_v.at[p], isems[p]).wait()

    def fire_g(p, j, b):
        return pltpu.async_copy(x_hbm.at[src_v.at[p, j]], rows_v.at[b],
                                gsems.at[b])

    def fire_s(p, j, b):
        return pltpu.async_copy(rows_v.at[b], acc_sh.at[dli_v.at[p, j]],
                                ssems.at[b], add=True)

    c_base = t * CH
    fire_idx(c_base, 0)

    zbuf = rows_v.at[0]

    def zrow(i, c):
        for k in range(D // L):
            zbuf[i, pl.ds(k * L, L)] = jnp.zeros((L,), jnp.float32)
        return c
    lax.fori_loop(0, CLEN, zrow, 0)
    base_acc = t * R_TILE
    for i in range(R_TILE // CLEN):
        pltpu.sync_copy(zbuf, acc_sh.at[pl.ds(base_acc + i * CLEN, CLEN)])
    rem = R_TILE % CLEN
    if rem:
        pltpu.sync_copy(zbuf.at[pl.ds(0, rem)],
                        acc_sh.at[pl.ds(base_acc + (R_TILE // CLEN) * CLEN, rem)])
    plsc.subcore_barrier()

    def scale(pv, j, b):
        rbuf = rows_v.at[b]

        def scale16(g, cc):
            v16 = vals_v[pv, pl.ds(j * CLEN + g * L, L)]
            for half in range(2):
                prods = []
                for e8 in range(8):
                    e = half * 8 + e8
                    r = g * L + e
                    v = v16[e]
                    for k in range(D // L):
                        prods.append((r, k, rbuf[r, pl.ds(k * L, L)] * v))
                for r, k, pr in prods:
                    rbuf[r, pl.ds(k * L, L)] = pr
            return cc
        lax.fori_loop(0, CLEN // L, scale16, 0)

    def ring_pass(subs, hooks):
        n = len(subs)
        gd = [None] * 3
        sd = [None] * 3
        for k in range(min(2, n)):
            p, j = subs[k]
            gd[k] = fire_g(p, j, k)
        for k, (p, j) in enumerate(subs):
            b = k % 3
            gd[b].wait()
            scale(p, j, b)
            if k + 2 < n:
                nb = (k + 2) % 3
                if sd[nb] is not None:
                    sd[nb].wait()
                    sd[nb] = None
                if k in hooks:
                    hooks[k]()
                pn, jn = subs[k + 2]
                gd[nb] = fire_g(pn, jn, nb)
            sd[b] = fire_s(p, j, b)
        for b in range(3):
            if sd[b] is not None:
                sd[b].wait()

    drain_idx(0)
    fire_idx(c_base + 1, 1)
    ring_pass([(0, j) for j in range(SUB)], {})
    fire_idx(c_base + 2, 0)

    def pair(i, carry):
        ca = c_base + 2 * i + 1
        drain_idx(1)
        subs = [(1, j) for j in range(SUB)] + [(0, j) for j in range(SUB)]
        hooks = {
            SUB - 3: lambda: drain_idx(0),
            SUB + 1: lambda: fire_idx(ca + 2, 1),
        }
        ring_pass(subs, hooks)
        fire_idx(ca + 3, 0)
        return carry
    lax.fori_loop(0, (CH - 1) // 2, pair, 0)

    drain_idx(1)
    drain_idx(0)

    plsc.subcore_barrier()
    pltpu.sync_copy(acc_sh.at[pl.ds(base_acc, R_TILE)],
                    out_hbm.at[pl.ds(s * HALF + base_acc, R_TILE)])


_layer = functools.partial(
    pl.kernel,
    out_type=jax.ShapeDtypeStruct((X_ROWS, D), jnp.float32),
    mesh=plsc.VectorSubcoreMesh(core_axis_name="c", subcore_axis_name="s",
                                num_cores=NC, num_subcores=NS),
    scratch_types=[
        pltpu.VMEM((2, SUB, CLEN), jnp.int32),
        pltpu.VMEM((2, SUB, CLEN), jnp.int32),
        pltpu.VMEM((2, K), jnp.float32),
        pltpu.VMEM((3, CLEN, D), jnp.float32),
        pltpu.VMEM_SHARED((HALF, D), jnp.float32),
        pltpu.SemaphoreType.DMA,
        pltpu.SemaphoreType.DMA,
        pltpu.SemaphoreType.DMA((3,)),
        pltpu.SemaphoreType.DMA((3,)),
    ],
    compiler_params=pltpu.CompilerParams(use_tc_tiling_on_sc=False),
)(_layer_body)


def kernel(adj_indices, adj_values, user_emb, item_emb):
    dst = adj_indices[0].astype(jnp.int32)
    src = adj_indices[1].astype(jnp.int32)
    vals = adj_values.astype(jnp.float32)

    pad = E_ALLOC - E
    dst = jnp.concatenate([dst, jnp.zeros((pad,), jnp.int32)])
    src = jnp.concatenate([src, jnp.zeros((pad,), jnp.int32)])
    vals = jnp.concatenate([vals, jnp.zeros((pad,), jnp.float32)])

    srcp = src + jnp.where(src >= N_USERS, HALF - N_USERS, 0).astype(jnp.int32)
    dl0 = jnp.where(dst < N_USERS, dst, TRASH).astype(jnp.int32)
    dl1 = jnp.where(dst >= N_USERS, dst - N_USERS, TRASH).astype(jnp.int32)

    src3 = srcp.reshape(NS * CH + 2, SUB, CLEN)
    dl4 = jnp.stack([dl0, dl1]).reshape(NC, NS * CH + 2, SUB, CLEN)
    vals2 = vals.reshape(NS * CH + 2, K)

    zpad = jnp.zeros((HALF - N_USERS, D), jnp.float32)
    x = jnp.concatenate([user_emb, zpad, item_emb, zpad], axis=0)

    acc = x
    for _ in range(N_LAYERS):
        x = _layer(x, src3, dl4, vals2)
        acc = acc + x
    out = acc * (1.0 / (N_LAYERS + 1))
    return (out[:N_USERS], out[HALF:HALF + N_ITEMS])
